# Initial kernel scaffold; baseline (speedup 1.0000x reference)
import jax
import jax.numpy as jnp
from jax.experimental import pallas as pl


def kernel(x, edge_index, edge_weight, W1, b1, W2, b2, W3, b3, W4, b4):
    raise NotImplementedError("write your pallas kernel here")


# speedup vs baseline: 6.2114x; 6.2114x over previous
import dataclasses
import functools

import jax
import jax.numpy as jnp
from jax import lax
from jax.experimental import pallas as pl
from jax.experimental.pallas import tpu as pltpu
from jax.experimental.pallas import tpu_sc as plsc

NC = 2
NS = 16
NW = NC * NS
L = 16
CH = 128
DEG_D = 128


def _vector_mesh():
    return plsc.VectorSubcoreMesh(core_axis_name="c", subcore_axis_name="s")


def _sc_compiler_params():
    cp = pltpu.CompilerParams()
    if "needs_layout_passes" in pltpu.CompilerParams.__dataclass_fields__:
        cp = dataclasses.replace(cp, needs_layout_passes=False)
    return cp


def _row_chunks(n, sid, do_copy):
    n_full = n // CH
    rem = n % CH
    n_tot = n_full + (1 if rem else 0)
    nt = -(-n_tot // NS)

    @pl.loop(0, nt)
    def _(t):
        q = t * NS + sid

        @pl.when(q < n_full)
        def _():
            do_copy(q * CH, CH)

        if rem:
            @pl.when(q == n_full)
            def _():
                do_copy(n_full * CH, rem)


def _sc_deg_partial(col, w, n, d):
    e = col.shape[0]
    n_chunks = e // CH
    n_loc = n_chunks // NW

    @functools.partial(
        pl.kernel,
        mesh=_vector_mesh(),
        out_type=jax.ShapeDtypeStruct((NC, n, d), jnp.float32),
        compiler_params=_sc_compiler_params(),
        scratch_types=[
            pltpu.VMEM((CH,), jnp.int32),
            pltpu.VMEM((CH,), jnp.float32),
            pltpu.VMEM((CH, d), jnp.float32),
            pltpu.VMEM_SHARED((n, d), jnp.float32),
        ],
    )
    def k(col_hbm, w_hbm, out_hbm, cidx, wv, rbuf, acc):
        cid = lax.axis_index("c")
        sid = lax.axis_index("s")
        wid = sid * NC + cid
        zero = jnp.zeros((L,), jnp.float32)

        @pl.loop(0, CH)
        def _(r):
            for j in range(d // L):
                rbuf[r, pl.ds(j * L, L)] = zero

        def _zero(off, size):
            off = pl.multiple_of(off, 8)
            pltpu.sync_copy(rbuf.at[pl.ds(0, size)], acc.at[pl.ds(off, size)])

        _row_chunks(n, sid, _zero)
        plsc.subcore_barrier()

        @pl.loop(0, n_loc)
        def _(t):
            base = (t * NW + wid) * CH
            pltpu.sync_copy(col_hbm.at[pl.ds(base, CH)], cidx)
            pltpu.sync_copy(w_hbm.at[pl.ds(base, CH)], wv)

            @pl.loop(0, CH)
            def _(r):
                s = plsc.load_gather(wv, [jnp.full((L,), r, jnp.int32)])
                for j in range(d // L):
                    rbuf[r, pl.ds(j * L, L)] = s

            pltpu.sync_copy(rbuf, acc.at[cidx], add=True)

        plsc.subcore_barrier()

        def _wb(off, size):
            off = pl.multiple_of(off, 8)
            pltpu.sync_copy(acc.at[pl.ds(off, size)],
                            out_hbm.at[cid, pl.ds(off, size)])

        _row_chunks(n, sid, _wb)

    return k(col, w)


def _sc_agg(g, row, col, w, n):
    e = row.shape[0]
    d = g.shape[1]
    n_chunks = e // CH
    n_loc = n_chunks // NW

    @functools.partial(
        pl.kernel,
        mesh=_vector_mesh(),
        out_type=jax.ShapeDtypeStruct((NC, n, d), jnp.float32),
        compiler_params=_sc_compiler_params(),
        scratch_types=[
            pltpu.VMEM((CH,), jnp.int32),
            pltpu.VMEM((CH,), jnp.int32),
            pltpu.VMEM((CH,), jnp.float32),
            pltpu.VMEM((CH, d), jnp.float32),
            pltpu.VMEM_SHARED((n, d), jnp.float32),
        ],
    )
    def k(g_hbm, row_hbm, col_hbm, w_hbm, out_hbm, ridx, cidx, wv, rbuf, acc):
        cid = lax.axis_index("c")
        sid = lax.axis_index("s")
        wid = sid * NC + cid
        zero = jnp.zeros((L,), jnp.float32)

        @pl.loop(0, CH)
        def _(r):
            for j in range(d // L):
                rbuf[r, pl.ds(j * L, L)] = zero

        def _zero(off, size):
            off = pl.multiple_of(off, 8)
            pltpu.sync_copy(rbuf.at[pl.ds(0, size)], acc.at[pl.ds(off, size)])

        _row_chunks(n, sid, _zero)
        plsc.subcore_barrier()

        @pl.loop(0, n_loc)
        def _(t):
            base = (t * NW + wid) * CH
            pltpu.sync_copy(row_hbm.at[pl.ds(base, CH)], ridx)
            pltpu.sync_copy(col_hbm.at[pl.ds(base, CH)], cidx)
            pltpu.sync_copy(w_hbm.at[pl.ds(base, CH)], wv)
            pltpu.sync_copy(g_hbm.at[ridx], rbuf)

            @pl.loop(0, CH)
            def _(r):
                s = plsc.load_gather(wv, [jnp.full((L,), r, jnp.int32)])
                for j in range(d // L):
                    rbuf[r, pl.ds(j * L, L)] = rbuf[r, pl.ds(j * L, L)] * s

            pltpu.sync_copy(rbuf, acc.at[cidx], add=True)

        plsc.subcore_barrier()

        def _wb(off, size):
            off = pl.multiple_of(off, 8)
            pltpu.sync_copy(acc.at[pl.ds(off, size)],
                            out_hbm.at[cid, pl.ds(off, size)])

        _row_chunks(n, sid, _wb)

    return k(g, row, col, w)


_R = 1000


def _tc_prep(x, w1, degp):
    n, d = x.shape

    def body(x_ref, w_ref, dp_ref, g_ref, dis_ref):
        d0 = dp_ref[0]
        d1 = dp_ref[1]
        nl = d0.shape[1]
        deg = (jnp.sum(d0, axis=1) + jnp.sum(d1, axis=1)) * (1.0 / nl) + 1.0
        good = deg > 0.0
        dis = jnp.where(good, lax.rsqrt(jnp.where(good, deg, 1.0)), 0.0)
        dis2 = jnp.broadcast_to(dis[:, None], (_R, d))
        h = jnp.dot(x_ref[...], w_ref[...], preferred_element_type=jnp.float32)
        g_ref[...] = h * dis2
        dis_ref[...] = dis2

    return pl.pallas_call(
        body,
        grid=(n // _R,),
        in_specs=[
            pl.BlockSpec((_R, d), lambda i: (i, 0)),
            pl.BlockSpec((d, d), lambda i: (0, 0)),
            pl.BlockSpec((2, _R, DEG_D), lambda i: (0, i, 0)),
        ],
        out_specs=[
            pl.BlockSpec((_R, d), lambda i: (i, 0)),
            pl.BlockSpec((_R, d), lambda i: (i, 0)),
        ],
        out_shape=[
            jax.ShapeDtypeStruct((n, d), jnp.float32),
            jax.ShapeDtypeStruct((n, d), jnp.float32),
        ],
    )(x, w1, degp)


def _tc_mid(agg, g, dis, b, w_next):
    n, d = g.shape

    def body(a_ref, g_ref, dis_ref, b_ref, w_ref, o_ref):
        s = a_ref[0] + a_ref[1] + g_ref[...]
        pre = s * dis_ref[...] + b_ref[...]
        t = jnp.maximum(pre, 0.01 * pre)
        o_ref[...] = jnp.dot(
            t, w_ref[...], preferred_element_type=jnp.float32) * dis_ref[...]

    return pl.pallas_call(
        body,
        grid=(n // _R,),
        in_specs=[
            pl.BlockSpec((2, _R, d), lambda i: (0, i, 0)),
            pl.BlockSpec((_R, d), lambda i: (i, 0)),
            pl.BlockSpec((_R, d), lambda i: (i, 0)),
            pl.BlockSpec((1, d), lambda i: (0, 0)),
            pl.BlockSpec((d, d), lambda i: (0, 0)),
        ],
        out_specs=pl.BlockSpec((_R, d), lambda i: (i, 0)),
        out_shape=jax.ShapeDtypeStruct((n, d), jnp.float32),
    )(agg, g, dis, b, w_next)


def _tc_final(agg, g, dis, b):
    n, d = g.shape

    def body(a_ref, g_ref, dis_ref, b_ref, o_ref):
        s = a_ref[0] + a_ref[1] + g_ref[...]
        o_ref[...] = s * dis_ref[...] + b_ref[...]

    return pl.pallas_call(
        body,
        grid=(n // _R,),
        in_specs=[
            pl.BlockSpec((2, _R, d), lambda i: (0, i, 0)),
            pl.BlockSpec((_R, d), lambda i: (i, 0)),
            pl.BlockSpec((_R, d), lambda i: (i, 0)),
            pl.BlockSpec((1, d), lambda i: (0, 0)),
        ],
        out_specs=pl.BlockSpec((_R, d), lambda i: (i, 0)),
        out_shape=jax.ShapeDtypeStruct((n, d), jnp.float32),
    )(agg, g, dis, b)


def kernel(x, edge_index, edge_weight, W1, b1, W2, b2, W3, b3, W4, b4):
    n, d = x.shape
    row = edge_index[0]
    col = edge_index[1]
    w = edge_weight.astype(jnp.float32)

    e = row.shape[0]
    pad = (-e) % (CH * NW)
    if pad:
        zi = jnp.zeros((pad,), row.dtype)
        row = jnp.concatenate([row, zi])
        col = jnp.concatenate([col, zi])
        w = jnp.concatenate([w, jnp.zeros((pad,), w.dtype)])

    degp = _sc_deg_partial(col, w, n, DEG_D)
    g, dis = _tc_prep(x, W1, degp)
    agg = _sc_agg(g, row, col, w, n)
    g = _tc_mid(agg, g, dis, b1.reshape(1, d), W2)
    agg = _sc_agg(g, row, col, w, n)
    g = _tc_mid(agg, g, dis, b2.reshape(1, d), W3)
    agg = _sc_agg(g, row, col, w, n)
    g = _tc_mid(agg, g, dis, b3.reshape(1, d), W4)
    agg = _sc_agg(g, row, col, w, n)
    return _tc_final(agg, g, dis, b4.reshape(1, d))
